# Initial kernel scaffold; baseline (speedup 1.0000x reference)
import jax
import jax.numpy as jnp
from jax.experimental import pallas as pl


def kernel(xyz, table):
    raise NotImplementedError("write your pallas kernel here")


# speedup vs baseline: 50.2249x; 50.2249x over previous
import numpy as np
import jax
import jax.numpy as jnp
from jax import lax
from jax.experimental import pallas as pl
from jax.experimental.pallas import tpu as pltpu
from jax.experimental.pallas import tpu_sc as plsc

N_LEVELS = 16
F = 2
LOG2_T = 19
T = 2 ** LOG2_T
BASE_RES = 16
PER_LEVEL_SCALE = 1.3819129
PRIMES = (1, 2654435761, 805459861)

NW = 32
C = 64
NCOL = 3 + N_LEVELS * F


def _levels():
    out = []
    for l in range(N_LEVELS):
        res = int(np.floor(BASE_RES * (PER_LEVEL_SCALE ** l)))
        stride = res + 1
        out.append((res, stride, stride ** 3 <= T, l * T))
    return out


LEVELS = _levels()


def _grid_coords(x, y, z, res):
    rf = jnp.float32(res)
    sx, sy, sz = x * rf, y * rf, z * rf
    ix = sx.astype(jnp.int32)
    iy = sy.astype(jnp.int32)
    iz = sz.astype(jnp.int32)
    return sx, sy, sz, ix, iy, iz


def _body(xs, ys, zs, tbl, out, xyz_loc, idx_buf, rows, obuf, sem):
    wid = lax.axis_index("s") * 2 + lax.axis_index("c")
    npts = xs.shape[0]
    per_w = npts // NW
    nblk = per_w // C
    iota = lax.iota(jnp.int32, 16)
    zero16 = jnp.zeros((16,), jnp.int32)
    one16 = jnp.ones((16,), jnp.int32)

    def block(b, carry):
        base = wid * per_w + b * C

        for d, src in enumerate((xs, ys, zs)):
            pltpu.sync_copy(src.at[pl.ds(base, C)], xyz_loc.at[d])

        def grp_a(g, c2):
            o = g * 16
            x = xyz_loc[0, pl.ds(o, 16)]
            y = xyz_loc[1, pl.ds(o, 16)]
            z = xyz_loc[2, pl.ds(o, 16)]
            for l, (res, stride, dense, lbase) in enumerate(LEVELS):
                _, _, _, ix, iy, iz = _grid_coords(x, y, z, res)
                if dense:
                    s2 = stride * stride
                    b000 = ix + iy * stride + iz * s2 + lbase
                    for corner in range(8):
                        off = ((corner & 1) + ((corner >> 1) & 1) * stride
                               + ((corner >> 2) & 1) * s2)
                        idx_buf[l * 8 + corner, pl.ds(o, 16)] = b000 + off
                else:
                    ux = ix.astype(jnp.uint32)
                    uy = iy.astype(jnp.uint32)
                    uz = iz.astype(jnp.uint32)
                    p1 = jnp.uint32(PRIMES[1])
                    p2 = jnp.uint32(PRIMES[2])
                    hy0 = uy * p1
                    hy1 = hy0 + p1
                    hz0 = uz * p2
                    hz1 = hz0 + p2
                    hx1 = ux + jnp.uint32(1)
                    mask = jnp.uint32(T - 1)
                    for corner in range(8):
                        hx = hx1 if (corner & 1) else ux
                        hy = hy1 if (corner & 2) else hy0
                        hz = hz1 if (corner & 4) else hz0
                        h = (hx ^ hy ^ hz) & mask
                        idx_buf[l * 8 + corner, pl.ds(o, 16)] = (
                            h.astype(jnp.int32) + lbase)
            return c2

        lax.fori_loop(0, C // 16, grp_a, 0)

        def fire(r, c2):
            pltpu.async_copy(tbl.at[idx_buf.at[r]], rows.at[r], sem)
            return c2

        lax.fori_loop(0, N_LEVELS * 8, fire, 0)

        def drain(r, c2):
            pltpu.make_async_copy(tbl.at[idx_buf.at[r]], rows.at[r], sem).wait()
            return c2

        lax.fori_loop(0, N_LEVELS * 8, drain, 0)

        def grp_c(g, c2):
            o = g * 16
            pv = iota + o
            pcol = pv * NCOL
            x = xyz_loc[0, pl.ds(o, 16)]
            y = xyz_loc[1, pl.ds(o, 16)]
            z = xyz_loc[2, pl.ds(o, 16)]
            plsc.store_scatter(obuf, [pcol], x * 2.0 - 1.0)
            plsc.store_scatter(obuf, [pcol + 1], y * 2.0 - 1.0)
            plsc.store_scatter(obuf, [pcol + 2], z * 2.0 - 1.0)
            for l, (res, stride, dense, lbase) in enumerate(LEVELS):
                sx, sy, sz, ix, iy, iz = _grid_coords(x, y, z, res)
                fx = sx - ix.astype(jnp.float32)
                fy = sy - iy.astype(jnp.float32)
                fz = sz - iz.astype(jnp.float32)
                gx, gy, gz = 1.0 - fx, 1.0 - fy, 1.0 - fz
                wxy = (gx * gy, fx * gy, gx * fy, fx * fy)
                acc0 = acc1 = None
                for corner in range(8):
                    wc = wxy[corner & 3] * (fz if (corner & 4) else gz)
                    rv = jnp.full((16,), l * 8 + corner, jnp.int32)
                    f0 = plsc.load_gather(rows, [rv, pv, zero16])
                    f1 = plsc.load_gather(rows, [rv, pv, one16])
                    if corner == 0:
                        acc0, acc1 = f0 * wc, f1 * wc
                    else:
                        acc0, acc1 = acc0 + f0 * wc, acc1 + f1 * wc
                plsc.store_scatter(obuf, [pcol + (3 + 2 * l)], acc0)
                plsc.store_scatter(obuf, [pcol + (4 + 2 * l)], acc1)
            return c2

        lax.fori_loop(0, C // 16, grp_c, 0)

        pltpu.sync_copy(obuf, out.at[pl.ds(base * NCOL, C * NCOL)])
        return carry

    lax.fori_loop(0, nblk, block, 0)


def kernel(xyz, table):
    n = xyz.shape[0]
    tbl = jnp.pad(table.reshape(N_LEVELS * T, F), ((0, 0), (0, 8 - F)))
    xs, ys, zs = xyz[:, 0], xyz[:, 1], xyz[:, 2]
    mesh = plsc.VectorSubcoreMesh(core_axis_name="c", subcore_axis_name="s")
    k = pl.kernel(
        _body,
        out_type=jax.ShapeDtypeStruct((n * NCOL,), jnp.float32),
        mesh=mesh,
        scratch_types=[
            pltpu.VMEM((3, C), jnp.float32),
            pltpu.VMEM((N_LEVELS * 8, C), jnp.int32),
            pltpu.VMEM((N_LEVELS * 8, C, 8), jnp.float32),
            pltpu.VMEM((C * NCOL,), jnp.float32),
            pltpu.SemaphoreType.DMA,
        ],
        compiler_params=pltpu.CompilerParams(
            needs_layout_passes=False, use_tc_tiling_on_sc=False),
    )
    return k(xs, ys, zs, tbl).reshape(n, NCOL)


# speedup vs baseline: 54.0842x; 1.0768x over previous
import numpy as np
import jax
import jax.numpy as jnp
from jax import lax
from jax.experimental import pallas as pl
from jax.experimental.pallas import tpu as pltpu
from jax.experimental.pallas import tpu_sc as plsc

N_LEVELS = 16
F = 2
LOG2_T = 19
T = 2 ** LOG2_T
BASE_RES = 16
PER_LEVEL_SCALE = 1.3819129
PRIMES = (1, 2654435761, 805459861)

NW = 32
C = 64
NCOL = 3 + N_LEVELS * F
NR = N_LEVELS * 8


def _levels():
    out = []
    for l in range(N_LEVELS):
        res = int(np.floor(BASE_RES * (PER_LEVEL_SCALE ** l)))
        stride = res + 1
        out.append((res, stride, stride ** 3 <= T, l * T))
    return out


LEVELS = _levels()


def _grid_coords(x, y, z, res):
    rf = jnp.float32(res)
    sx, sy, sz = x * rf, y * rf, z * rf
    ix = sx.astype(jnp.int32)
    iy = sy.astype(jnp.int32)
    iz = sz.astype(jnp.int32)
    return sx, sy, sz, ix, iy, iz


def _body(xyzf, tbl, out, xyz_loc, idx_buf, fv_buf, rows, obuf, sem):
    wid = lax.axis_index("s") * 2 + lax.axis_index("c")
    npts = xyzf.shape[0] // 3
    per_w = npts // NW
    nblk = per_w // C
    iota = lax.iota(jnp.int32, 16)

    def block(b, carry):
        base = wid * per_w + b * C

        pltpu.sync_copy(xyzf.at[pl.ds(base * 3, C * 3)], xyz_loc)

        def grp_a(g, c2):
            o = g * 16
            p3 = (iota + o) * 3
            x = plsc.load_gather(xyz_loc, [p3])
            y = plsc.load_gather(xyz_loc, [p3 + 1])
            z = plsc.load_gather(xyz_loc, [p3 + 2])
            for l, (res, stride, dense, lbase) in enumerate(LEVELS):
                _, _, _, ix, iy, iz = _grid_coords(x, y, z, res)
                if dense:
                    s2 = stride * stride
                    b000 = ix + iy * stride + iz * s2 + lbase
                    for corner in range(8):
                        off = ((corner & 1) + ((corner >> 1) & 1) * stride
                               + ((corner >> 2) & 1) * s2)
                        e = b000 + off
                        idx_buf[l * 8 + corner, pl.ds(o, 16)] = e >> 2
                        fv_buf[l * 8 + corner, pl.ds(o, 16)] = (e & 3) << 1
                else:
                    ux = ix.astype(jnp.uint32)
                    uy = iy.astype(jnp.uint32)
                    uz = iz.astype(jnp.uint32)
                    p1 = jnp.uint32(PRIMES[1])
                    p2 = jnp.uint32(PRIMES[2])
                    hy0 = uy * p1
                    hy1 = hy0 + p1
                    hz0 = uz * p2
                    hz1 = hz0 + p2
                    hx1 = ux + jnp.uint32(1)
                    mask = jnp.uint32(T - 1)
                    for corner in range(8):
                        hx = hx1 if (corner & 1) else ux
                        hy = hy1 if (corner & 2) else hy0
                        hz = hz1 if (corner & 4) else hz0
                        h = (hx ^ hy ^ hz) & mask
                        e = h.astype(jnp.int32) + lbase
                        idx_buf[l * 8 + corner, pl.ds(o, 16)] = e >> 2
                        fv_buf[l * 8 + corner, pl.ds(o, 16)] = (e & 3) << 1
            return c2

        lax.fori_loop(0, C // 16, grp_a, 0)

        def fire(r, c2):
            pltpu.async_copy(tbl.at[idx_buf.at[r]], rows.at[r], sem)
            return c2

        lax.fori_loop(0, NR, fire, 0)

        def drain(r, c2):
            pltpu.make_async_copy(tbl.at[idx_buf.at[r]], rows.at[r], sem).wait()
            return c2

        lax.fori_loop(0, NR, drain, 0)

        def grp_c(g, c2):
            o = g * 16
            pv = iota + o
            p3 = pv * 3
            pcol = pv * NCOL
            x = plsc.load_gather(xyz_loc, [p3])
            y = plsc.load_gather(xyz_loc, [p3 + 1])
            z = plsc.load_gather(xyz_loc, [p3 + 2])
            plsc.store_scatter(obuf, [pcol], x * 2.0 - 1.0)
            plsc.store_scatter(obuf, [pcol + 1], y * 2.0 - 1.0)
            plsc.store_scatter(obuf, [pcol + 2], z * 2.0 - 1.0)
            for l, (res, stride, dense, lbase) in enumerate(LEVELS):
                sx, sy, sz, ix, iy, iz = _grid_coords(x, y, z, res)
                fx = sx - ix.astype(jnp.float32)
                fy = sy - iy.astype(jnp.float32)
                fz = sz - iz.astype(jnp.float32)
                gx, gy, gz = 1.0 - fx, 1.0 - fy, 1.0 - fz
                wxy = (gx * gy, fx * gy, gx * fy, fx * fy)
                acc0 = acc1 = None
                for corner in range(8):
                    wc = wxy[corner & 3] * (fz if (corner & 4) else gz)
                    rv = jnp.full((16,), l * 8 + corner, jnp.int32)
                    fv = fv_buf[l * 8 + corner, pl.ds(o, 16)]
                    f0 = plsc.load_gather(rows, [rv, pv, fv])
                    f1 = plsc.load_gather(rows, [rv, pv, fv + 1])
                    if corner == 0:
                        acc0, acc1 = f0 * wc, f1 * wc
                    else:
                        acc0, acc1 = acc0 + f0 * wc, acc1 + f1 * wc
                plsc.store_scatter(obuf, [pcol + (3 + 2 * l)], acc0)
                plsc.store_scatter(obuf, [pcol + (4 + 2 * l)], acc1)
            return c2

        lax.fori_loop(0, C // 16, grp_c, 0)

        pltpu.sync_copy(obuf, out.at[pl.ds(base * NCOL, C * NCOL)])
        return carry

    lax.fori_loop(0, nblk, block, 0)


def kernel(xyz, table):
    n = xyz.shape[0]
    tbl = table.reshape(N_LEVELS * T * F // 8, 8)
    xyzf = xyz.reshape(n * 3)
    mesh = plsc.VectorSubcoreMesh(core_axis_name="c", subcore_axis_name="s")
    k = pl.kernel(
        _body,
        out_type=jax.ShapeDtypeStruct((n * NCOL,), jnp.float32),
        mesh=mesh,
        scratch_types=[
            pltpu.VMEM((C * 3,), jnp.float32),
            pltpu.VMEM((NR, C), jnp.int32),
            pltpu.VMEM((NR, C), jnp.int32),
            pltpu.VMEM((NR, C, 8), jnp.float32),
            pltpu.VMEM((C * NCOL,), jnp.float32),
            pltpu.SemaphoreType.DMA,
        ],
        compiler_params=pltpu.CompilerParams(
            needs_layout_passes=False, use_tc_tiling_on_sc=False),
    )
    return k(xyzf, tbl).reshape(n, NCOL)


# speedup vs baseline: 114.1452x; 2.1105x over previous
import numpy as np
import jax
import jax.numpy as jnp
from jax import lax
from jax.experimental import pallas as pl
from jax.experimental.pallas import tpu as pltpu
from jax.experimental.pallas import tpu_sc as plsc

N_LEVELS = 16
F = 2
LOG2_T = 19
T = 2 ** LOG2_T
BASE_RES = 16
PER_LEVEL_SCALE = 1.3819129
PRIMES = (1, 2654435761, 805459861)

NW = 32
C = 64
NCOL = 3 + N_LEVELS * F
NR = N_LEVELS * 8


def _levels():
    out = []
    for l in range(N_LEVELS):
        res = int(np.floor(BASE_RES * (PER_LEVEL_SCALE ** l)))
        stride = res + 1
        out.append((res, stride, stride ** 3 <= T, l * T))
    return out


LEVELS = _levels()


def _grid_coords(x, y, z, res):
    rf = jnp.float32(res)
    sx, sy, sz = x * rf, y * rf, z * rf
    ix = sx.astype(jnp.int32)
    iy = sy.astype(jnp.int32)
    iz = sz.astype(jnp.int32)
    return sx, sy, sz, ix, iy, iz


RELAYOUT_CH = 16384


def _relayout_body(tsrc, tdst, src_loc, dst_loc):
    wid = lax.axis_index("s") * 2 + lax.axis_index("c")
    span = tsrc.shape[0] // NW
    base = wid * span
    iota = lax.iota(jnp.int32, 16)
    io2 = iota * 2

    def chunk(c, carry):
        off = base + c * RELAYOUT_CH
        pltpu.sync_copy(tsrc.at[pl.ds(off, RELAYOUT_CH)], src_loc)

        def tile(t, c2):
            tb = t * 256
            for k in range(8):
                f0 = src_loc[pl.ds(tb + k * 16, 16)]
                f1 = src_loc[pl.ds(tb + 128 + k * 16, 16)]
                di = io2 + (tb + k * 32)
                plsc.store_scatter(dst_loc, [di], f0)
                plsc.store_scatter(dst_loc, [di + 1], f1)
            return c2

        lax.fori_loop(0, RELAYOUT_CH // 256, tile, 0)
        pltpu.sync_copy(dst_loc, tdst.at[pl.ds(off, RELAYOUT_CH)])
        return carry

    lax.fori_loop(0, span // RELAYOUT_CH, chunk, 0)


def _body(xyzf, tbl, out, xyz_loc, idx_buf, fv_buf, rows, obuf, sem):
    wid = lax.axis_index("s") * 2 + lax.axis_index("c")
    npts = xyzf.shape[0] // 3
    per_w = npts // NW
    nblk = per_w // C
    iota = lax.iota(jnp.int32, 16)

    def block(b, carry):
        base = wid * per_w + b * C

        pltpu.sync_copy(xyzf.at[pl.ds(base * 3, C * 3)], xyz_loc)

        def grp_a(g, c2):
            o = g * 16
            p3 = (iota + o) * 3
            x = plsc.load_gather(xyz_loc, [p3])
            y = plsc.load_gather(xyz_loc, [p3 + 1])
            z = plsc.load_gather(xyz_loc, [p3 + 2])
            for l, (res, stride, dense, lbase) in enumerate(LEVELS):
                _, _, _, ix, iy, iz = _grid_coords(x, y, z, res)
                if dense:
                    s2 = stride * stride
                    b000 = ix + iy * stride + iz * s2 + lbase
                    for corner in range(8):
                        off = ((corner & 1) + ((corner >> 1) & 1) * stride
                               + ((corner >> 2) & 1) * s2)
                        e = b000 + off
                        idx_buf[l * 8 + corner, pl.ds(o, 16)] = e >> 2
                        fv_buf[l * 8 + corner, pl.ds(o, 16)] = (e & 3) << 1
                else:
                    ux = ix.astype(jnp.uint32)
                    uy = iy.astype(jnp.uint32)
                    uz = iz.astype(jnp.uint32)
                    p1 = jnp.uint32(PRIMES[1])
                    p2 = jnp.uint32(PRIMES[2])
                    hy0 = uy * p1
                    hy1 = hy0 + p1
                    hz0 = uz * p2
                    hz1 = hz0 + p2
                    hx1 = ux + jnp.uint32(1)
                    mask = jnp.uint32(T - 1)
                    for corner in range(8):
                        hx = hx1 if (corner & 1) else ux
                        hy = hy1 if (corner & 2) else hy0
                        hz = hz1 if (corner & 4) else hz0
                        h = (hx ^ hy ^ hz) & mask
                        e = h.astype(jnp.int32) + lbase
                        idx_buf[l * 8 + corner, pl.ds(o, 16)] = e >> 2
                        fv_buf[l * 8 + corner, pl.ds(o, 16)] = (e & 3) << 1
            return c2

        lax.fori_loop(0, C // 16, grp_a, 0)

        def fire(r, c2):
            pltpu.async_copy(tbl.at[idx_buf.at[r]], rows.at[r], sem)
            return c2

        lax.fori_loop(0, NR, fire, 0)

        def drain(r, c2):
            pltpu.make_async_copy(tbl.at[idx_buf.at[r]], rows.at[r], sem).wait()
            return c2

        lax.fori_loop(0, NR, drain, 0)

        def grp_c(g, c2):
            o = g * 16
            pv = iota + o
            p3 = pv * 3
            pcol = pv * NCOL
            x = plsc.load_gather(xyz_loc, [p3])
            y = plsc.load_gather(xyz_loc, [p3 + 1])
            z = plsc.load_gather(xyz_loc, [p3 + 2])
            plsc.store_scatter(obuf, [pcol], x * 2.0 - 1.0)
            plsc.store_scatter(obuf, [pcol + 1], y * 2.0 - 1.0)
            plsc.store_scatter(obuf, [pcol + 2], z * 2.0 - 1.0)
            for l, (res, stride, dense, lbase) in enumerate(LEVELS):
                sx, sy, sz, ix, iy, iz = _grid_coords(x, y, z, res)
                fx = sx - ix.astype(jnp.float32)
                fy = sy - iy.astype(jnp.float32)
                fz = sz - iz.astype(jnp.float32)
                gx, gy, gz = 1.0 - fx, 1.0 - fy, 1.0 - fz
                wxy = (gx * gy, fx * gy, gx * fy, fx * fy)
                acc0 = acc1 = None
                for corner in range(8):
                    wc = wxy[corner & 3] * (fz if (corner & 4) else gz)
                    rv = jnp.full((16,), l * 8 + corner, jnp.int32)
                    fv = fv_buf[l * 8 + corner, pl.ds(o, 16)]
                    f0 = plsc.load_gather(rows, [rv, pv, fv])
                    f1 = plsc.load_gather(rows, [rv, pv, fv + 1])
                    if corner == 0:
                        acc0, acc1 = f0 * wc, f1 * wc
                    else:
                        acc0, acc1 = acc0 + f0 * wc, acc1 + f1 * wc
                plsc.store_scatter(obuf, [pcol + (3 + 2 * l)], acc0)
                plsc.store_scatter(obuf, [pcol + (4 + 2 * l)], acc1)
            return c2

        lax.fori_loop(0, C // 16, grp_c, 0)

        pltpu.sync_copy(obuf, out.at[pl.ds(base * NCOL, C * NCOL)])
        return carry

    lax.fori_loop(0, nblk, block, 0)


def kernel(xyz, table):
    n = xyz.shape[0]
    nw = N_LEVELS * T * F
    tnative = (table.reshape(N_LEVELS, T // 128, 128, F)
               .transpose(0, 1, 3, 2).reshape(nw))
    xyzf = xyz.reshape(n * 3)
    mesh = plsc.VectorSubcoreMesh(core_axis_name="c", subcore_axis_name="s")
    k1 = pl.kernel(
        _relayout_body,
        out_type=jax.ShapeDtypeStruct((nw,), jnp.float32),
        mesh=mesh,
        scratch_types=[
            pltpu.VMEM((RELAYOUT_CH,), jnp.float32),
            pltpu.VMEM((RELAYOUT_CH,), jnp.float32),
        ],
        compiler_params=pltpu.CompilerParams(
            needs_layout_passes=False, use_tc_tiling_on_sc=False),
    )
    tbl = k1(tnative).reshape(nw // 8, 8)
    k = pl.kernel(
        _body,
        out_type=jax.ShapeDtypeStruct((n * NCOL,), jnp.float32),
        mesh=mesh,
        scratch_types=[
            pltpu.VMEM((C * 3,), jnp.float32),
            pltpu.VMEM((NR, C), jnp.int32),
            pltpu.VMEM((NR, C), jnp.int32),
            pltpu.VMEM((NR, C, 8), jnp.float32),
            pltpu.VMEM((C * NCOL,), jnp.float32),
            pltpu.SemaphoreType.DMA,
        ],
        compiler_params=pltpu.CompilerParams(
            needs_layout_passes=False, use_tc_tiling_on_sc=False),
    )
    return k(xyzf, tbl).reshape(n, NCOL)


# speedup vs baseline: 144.1777x; 1.2631x over previous
import numpy as np
import jax
import jax.numpy as jnp
from jax import lax
from jax.experimental import pallas as pl
from jax.experimental.pallas import tpu as pltpu
from jax.experimental.pallas import tpu_sc as plsc

N_LEVELS = 16
F = 2
LOG2_T = 19
T = 2 ** LOG2_T
BASE_RES = 16
PER_LEVEL_SCALE = 1.3819129
PRIMES = (1, 2654435761, 805459861)

NW = 32
C = 64
NCOL = 3 + N_LEVELS * F
NR = N_LEVELS * 8


def _levels():
    out = []
    for l in range(N_LEVELS):
        res = int(np.floor(BASE_RES * (PER_LEVEL_SCALE ** l)))
        stride = res + 1
        out.append((res, stride, stride ** 3 <= T, l * T))
    return out


LEVELS = _levels()


def _grid_coords(x, y, z, res):
    rf = jnp.float32(res)
    sx, sy, sz = x * rf, y * rf, z * rf
    ix = sx.astype(jnp.int32)
    iy = sy.astype(jnp.int32)
    iz = sz.astype(jnp.int32)
    return sx, sy, sz, ix, iy, iz


RELAYOUT_CH = 16384


def _relayout_body(tsrc, tdst, src_loc, dst_loc):
    wid = lax.axis_index("s") * 2 + lax.axis_index("c")
    span = tsrc.shape[0] // NW
    base = wid * span
    iota = lax.iota(jnp.int32, 16)
    io2 = iota * 2

    def chunk(c, carry):
        off = base + c * RELAYOUT_CH
        pltpu.sync_copy(tsrc.at[pl.ds(off, RELAYOUT_CH)], src_loc)

        def tile(t, c2):
            tb = t * 256
            for k in range(8):
                f0 = src_loc[pl.ds(tb + k * 16, 16)]
                f1 = src_loc[pl.ds(tb + 128 + k * 16, 16)]
                di = io2 + (tb + k * 32)
                plsc.store_scatter(dst_loc, [di], f0)
                plsc.store_scatter(dst_loc, [di + 1], f1)
            return c2

        lax.fori_loop(0, RELAYOUT_CH // 256, tile, 0)
        pltpu.sync_copy(dst_loc, tdst.at[pl.ds(off, RELAYOUT_CH)])
        return carry

    lax.fori_loop(0, span // RELAYOUT_CH, chunk, 0)


def _body(xyzf, tbl, out, xyz_loc, idx_buf, fv_buf, rows, obuf, sems):
    wid = lax.axis_index("s") * 2 + lax.axis_index("c")
    npts = xyzf.shape[0] // 3
    per_w = npts // NW
    nblk = per_w // C
    iota = lax.iota(jnp.int32, 16)

    def block(b, carry):
        base = wid * per_w + b * C

        pltpu.sync_copy(xyzf.at[pl.ds(base * 3, C * 3)], xyz_loc)

        for l, (res, stride, dense, lbase) in enumerate(LEVELS):

            def grp_a(g, c2, res=res, stride=stride, dense=dense,
                      lbase=lbase, l=l):
                o = g * 16
                p3 = (iota + o) * 3
                x = plsc.load_gather(xyz_loc, [p3])
                y = plsc.load_gather(xyz_loc, [p3 + 1])
                z = plsc.load_gather(xyz_loc, [p3 + 2])
                _, _, _, ix, iy, iz = _grid_coords(x, y, z, res)
                if dense:
                    s2 = stride * stride
                    b000 = ix + iy * stride + iz * s2 + lbase
                    for corner in range(8):
                        off = ((corner & 1) + ((corner >> 1) & 1) * stride
                               + ((corner >> 2) & 1) * s2)
                        e = b000 + off
                        idx_buf[l * 8 + corner, pl.ds(o, 16)] = e >> 2
                        fv_buf[l * 8 + corner, pl.ds(o, 16)] = (e & 3) << 1
                else:
                    ux = ix.astype(jnp.uint32)
                    uy = iy.astype(jnp.uint32)
                    uz = iz.astype(jnp.uint32)
                    p1 = jnp.uint32(PRIMES[1])
                    p2 = jnp.uint32(PRIMES[2])
                    hy0 = uy * p1
                    hy1 = hy0 + p1
                    hz0 = uz * p2
                    hz1 = hz0 + p2
                    hx1 = ux + jnp.uint32(1)
                    mask = jnp.uint32(T - 1)
                    for corner in range(8):
                        hx = hx1 if (corner & 1) else ux
                        hy = hy1 if (corner & 2) else hy0
                        hz = hz1 if (corner & 4) else hz0
                        h = (hx ^ hy ^ hz) & mask
                        e = h.astype(jnp.int32) + lbase
                        idx_buf[l * 8 + corner, pl.ds(o, 16)] = e >> 2
                        fv_buf[l * 8 + corner, pl.ds(o, 16)] = (e & 3) << 1
                return c2

            lax.fori_loop(0, C // 16, grp_a, 0)
            for corner in range(8):
                r = l * 8 + corner
                pltpu.async_copy(tbl.at[idx_buf.at[r]], rows.at[r], sems.at[l])

        for l, (res, stride, dense, lbase) in enumerate(LEVELS):
            for corner in range(8):
                r = l * 8 + corner
                pltpu.make_async_copy(
                    tbl.at[idx_buf.at[r]], rows.at[r], sems.at[l]).wait()

            def grp_c(g, c2, res=res, l=l):
                o = g * 16
                pv = iota + o
                p3 = pv * 3
                pcol = pv * NCOL
                x = plsc.load_gather(xyz_loc, [p3])
                y = plsc.load_gather(xyz_loc, [p3 + 1])
                z = plsc.load_gather(xyz_loc, [p3 + 2])
                sx, sy, sz, ix, iy, iz = _grid_coords(x, y, z, res)
                fx = sx - ix.astype(jnp.float32)
                fy = sy - iy.astype(jnp.float32)
                fz = sz - iz.astype(jnp.float32)
                gx, gy, gz = 1.0 - fx, 1.0 - fy, 1.0 - fz
                wxy = (gx * gy, fx * gy, gx * fy, fx * fy)
                acc0 = acc1 = None
                for corner in range(8):
                    wc = wxy[corner & 3] * (fz if (corner & 4) else gz)
                    rv = jnp.full((16,), l * 8 + corner, jnp.int32)
                    fv = fv_buf[l * 8 + corner, pl.ds(o, 16)]
                    f0 = plsc.load_gather(rows, [rv, pv, fv])
                    f1 = plsc.load_gather(rows, [rv, pv, fv + 1])
                    if corner == 0:
                        acc0, acc1 = f0 * wc, f1 * wc
                    else:
                        acc0, acc1 = acc0 + f0 * wc, acc1 + f1 * wc
                plsc.store_scatter(obuf, [pcol + (3 + 2 * l)], acc0)
                plsc.store_scatter(obuf, [pcol + (4 + 2 * l)], acc1)
                return c2

            lax.fori_loop(0, C // 16, grp_c, 0)

        def grp_x(g, c2):
            o = g * 16
            pv = iota + o
            p3 = pv * 3
            pcol = pv * NCOL
            x = plsc.load_gather(xyz_loc, [p3])
            y = plsc.load_gather(xyz_loc, [p3 + 1])
            z = plsc.load_gather(xyz_loc, [p3 + 2])
            plsc.store_scatter(obuf, [pcol], x * 2.0 - 1.0)
            plsc.store_scatter(obuf, [pcol + 1], y * 2.0 - 1.0)
            plsc.store_scatter(obuf, [pcol + 2], z * 2.0 - 1.0)
            return c2

        lax.fori_loop(0, C // 16, grp_x, 0)

        pltpu.sync_copy(obuf, out.at[pl.ds(base * NCOL, C * NCOL)])
        return carry

    lax.fori_loop(0, nblk, block, 0)


def kernel(xyz, table):
    n = xyz.shape[0]
    nw = N_LEVELS * T * F
    tnative = (table.reshape(N_LEVELS, T // 128, 128, F)
               .transpose(0, 1, 3, 2).reshape(nw))
    xyzf = xyz.reshape(n * 3)
    mesh = plsc.VectorSubcoreMesh(core_axis_name="c", subcore_axis_name="s")
    k1 = pl.kernel(
        _relayout_body,
        out_type=jax.ShapeDtypeStruct((nw,), jnp.float32),
        mesh=mesh,
        scratch_types=[
            pltpu.VMEM((RELAYOUT_CH,), jnp.float32),
            pltpu.VMEM((RELAYOUT_CH,), jnp.float32),
        ],
        compiler_params=pltpu.CompilerParams(
            needs_layout_passes=False, use_tc_tiling_on_sc=False),
    )
    tbl = k1(tnative).reshape(nw // 8, 8)
    k = pl.kernel(
        _body,
        out_type=jax.ShapeDtypeStruct((n * NCOL,), jnp.float32),
        mesh=mesh,
        scratch_types=[
            pltpu.VMEM((C * 3,), jnp.float32),
            pltpu.VMEM((NR, C), jnp.int32),
            pltpu.VMEM((NR, C), jnp.int32),
            pltpu.VMEM((NR, C, 8), jnp.float32),
            pltpu.VMEM((C * NCOL,), jnp.float32),
            pltpu.SemaphoreType.DMA((N_LEVELS,)),
        ],
        compiler_params=pltpu.CompilerParams(
            needs_layout_passes=False, use_tc_tiling_on_sc=False),
    )
    return k(xyzf, tbl).reshape(n, NCOL)


# speedup vs baseline: 145.4622x; 1.0089x over previous
import numpy as np
import jax
import jax.numpy as jnp
from jax import lax
from jax.experimental import pallas as pl
from jax.experimental.pallas import tpu as pltpu
from jax.experimental.pallas import tpu_sc as plsc

N_LEVELS = 16
F = 2
LOG2_T = 19
T = 2 ** LOG2_T
BASE_RES = 16
PER_LEVEL_SCALE = 1.3819129
PRIMES = (1, 2654435761, 805459861)

NW = 32
C = 64
NCOL = 3 + N_LEVELS * F
NR = N_LEVELS * 8


def _levels():
    out = []
    for l in range(N_LEVELS):
        res = int(np.floor(BASE_RES * (PER_LEVEL_SCALE ** l)))
        stride = res + 1
        out.append((res, stride, stride ** 3 <= T, l * T))
    return out


LEVELS = _levels()


def _grid_coords(x, y, z, res):
    rf = jnp.float32(res)
    sx, sy, sz = x * rf, y * rf, z * rf
    ix = sx.astype(jnp.int32)
    iy = sy.astype(jnp.int32)
    iz = sz.astype(jnp.int32)
    return sx, sy, sz, ix, iy, iz


RELAYOUT_CH = 16384


def _relayout_body(tsrc, tdst, src_loc, dst_loc):
    wid = lax.axis_index("s") * 2 + lax.axis_index("c")
    span = tsrc.shape[0] // NW
    base = wid * span
    iota = lax.iota(jnp.int32, 16)
    io2 = iota * 2

    def chunk(c, carry):
        off = base + c * RELAYOUT_CH
        pltpu.sync_copy(tsrc.at[pl.ds(off, RELAYOUT_CH)], src_loc)

        def tile(t, c2):
            tb = t * 256
            for k in range(8):
                f0 = src_loc[pl.ds(tb + k * 16, 16)]
                f1 = src_loc[pl.ds(tb + 128 + k * 16, 16)]
                di = io2 + (tb + k * 32)
                plsc.store_scatter(dst_loc, [di], f0)
                plsc.store_scatter(dst_loc, [di + 1], f1)
            return c2

        lax.fori_loop(0, RELAYOUT_CH // 256, tile, 0)
        pltpu.sync_copy(dst_loc, tdst.at[pl.ds(off, RELAYOUT_CH)])
        return carry

    lax.fori_loop(0, span // RELAYOUT_CH, chunk, 0)


def _body(xyzf, tbl, out, xyz_loc, idx_buf, fv_buf, w_buf, rows, obuf, sems):
    wid = lax.axis_index("s") * 2 + lax.axis_index("c")
    npts = xyzf.shape[0] // 3
    per_w = npts // NW
    nblk = per_w // C
    iota = lax.iota(jnp.int32, 16)

    def block(b, carry):
        base = wid * per_w + b * C

        pltpu.sync_copy(xyzf.at[pl.ds(base * 3, C * 3)], xyz_loc)

        for l, (res, stride, dense, lbase) in enumerate(LEVELS):

            def grp_a(g, c2, res=res, stride=stride, dense=dense,
                      lbase=lbase, l=l):
                for so in (0, 16):
                    o = g * 32 + so
                    p3 = (iota + o) * 3
                    x = plsc.load_gather(xyz_loc, [p3])
                    y = plsc.load_gather(xyz_loc, [p3 + 1])
                    z = plsc.load_gather(xyz_loc, [p3 + 2])
                    sx, sy, sz, ix, iy, iz = _grid_coords(x, y, z, res)
                    w_buf[l * 3, pl.ds(o, 16)] = sx - ix.astype(jnp.float32)
                    w_buf[l * 3 + 1, pl.ds(o, 16)] = sy - iy.astype(jnp.float32)
                    w_buf[l * 3 + 2, pl.ds(o, 16)] = sz - iz.astype(jnp.float32)
                    if dense:
                        s2 = stride * stride
                        b000 = ix + iy * stride + iz * s2 + lbase
                        for corner in range(8):
                            off = ((corner & 1) + ((corner >> 1) & 1) * stride
                                   + ((corner >> 2) & 1) * s2)
                            e = b000 + off
                            idx_buf[l * 8 + corner, pl.ds(o, 16)] = e >> 2
                            fv_buf[l * 8 + corner, pl.ds(o, 16)] = (e & 3) << 1
                    else:
                        ux = ix.astype(jnp.uint32)
                        uy = iy.astype(jnp.uint32)
                        uz = iz.astype(jnp.uint32)
                        p1 = jnp.uint32(PRIMES[1])
                        p2 = jnp.uint32(PRIMES[2])
                        hy0 = uy * p1
                        hy1 = hy0 + p1
                        hz0 = uz * p2
                        hz1 = hz0 + p2
                        hx1 = ux + jnp.uint32(1)
                        mask = jnp.uint32(T - 1)
                        for corner in range(8):
                            hx = hx1 if (corner & 1) else ux
                            hy = hy1 if (corner & 2) else hy0
                            hz = hz1 if (corner & 4) else hz0
                            h = (hx ^ hy ^ hz) & mask
                            e = h.astype(jnp.int32) + lbase
                            idx_buf[l * 8 + corner, pl.ds(o, 16)] = e >> 2
                            fv_buf[l * 8 + corner, pl.ds(o, 16)] = (e & 3) << 1
                return c2

            lax.fori_loop(0, C // 32, grp_a, 0)
            for corner in range(8):
                r = l * 8 + corner
                pltpu.async_copy(tbl.at[idx_buf.at[r]], rows.at[r], sems.at[l])

        for l, (res, stride, dense, lbase) in enumerate(LEVELS):
            for corner in range(8):
                r = l * 8 + corner
                pltpu.make_async_copy(
                    tbl.at[idx_buf.at[r]], rows.at[r], sems.at[l]).wait()

            def grp_c(g, c2, l=l):
                for so in (0, 16):
                    o = g * 32 + so
                    pv = iota + o
                    pcol = pv * NCOL
                    fx = w_buf[l * 3, pl.ds(o, 16)]
                    fy = w_buf[l * 3 + 1, pl.ds(o, 16)]
                    fz = w_buf[l * 3 + 2, pl.ds(o, 16)]
                    gx, gy, gz = 1.0 - fx, 1.0 - fy, 1.0 - fz
                    wxy = (gx * gy, fx * gy, gx * fy, fx * fy)
                    acc0 = acc1 = None
                    for corner in range(8):
                        wc = wxy[corner & 3] * (fz if (corner & 4) else gz)
                        rv = jnp.full((16,), l * 8 + corner, jnp.int32)
                        fv = fv_buf[l * 8 + corner, pl.ds(o, 16)]
                        f0 = plsc.load_gather(rows, [rv, pv, fv])
                        f1 = plsc.load_gather(rows, [rv, pv, fv + 1])
                        if corner == 0:
                            acc0, acc1 = f0 * wc, f1 * wc
                        else:
                            acc0, acc1 = acc0 + f0 * wc, acc1 + f1 * wc
                    plsc.store_scatter(obuf, [pcol + (3 + 2 * l)], acc0)
                    plsc.store_scatter(obuf, [pcol + (4 + 2 * l)], acc1)
                return c2

            lax.fori_loop(0, C // 32, grp_c, 0)

        def grp_x(g, c2):
            o = g * 16
            pv = iota + o
            p3 = pv * 3
            pcol = pv * NCOL
            x = plsc.load_gather(xyz_loc, [p3])
            y = plsc.load_gather(xyz_loc, [p3 + 1])
            z = plsc.load_gather(xyz_loc, [p3 + 2])
            plsc.store_scatter(obuf, [pcol], x * 2.0 - 1.0)
            plsc.store_scatter(obuf, [pcol + 1], y * 2.0 - 1.0)
            plsc.store_scatter(obuf, [pcol + 2], z * 2.0 - 1.0)
            return c2

        lax.fori_loop(0, C // 16, grp_x, 0)

        pltpu.sync_copy(obuf, out.at[pl.ds(base * NCOL, C * NCOL)])
        return carry

    lax.fori_loop(0, nblk, block, 0)


def kernel(xyz, table):
    n = xyz.shape[0]
    nw = N_LEVELS * T * F
    tnative = (table.reshape(N_LEVELS, T // 128, 128, F)
               .transpose(0, 1, 3, 2).reshape(nw))
    xyzf = xyz.reshape(n * 3)
    mesh = plsc.VectorSubcoreMesh(core_axis_name="c", subcore_axis_name="s")
    k1 = pl.kernel(
        _relayout_body,
        out_type=jax.ShapeDtypeStruct((nw,), jnp.float32),
        mesh=mesh,
        scratch_types=[
            pltpu.VMEM((RELAYOUT_CH,), jnp.float32),
            pltpu.VMEM((RELAYOUT_CH,), jnp.float32),
        ],
        compiler_params=pltpu.CompilerParams(
            needs_layout_passes=False, use_tc_tiling_on_sc=False),
    )
    tbl = k1(tnative).reshape(nw // 8, 8)
    k = pl.kernel(
        _body,
        out_type=jax.ShapeDtypeStruct((n * NCOL,), jnp.float32),
        mesh=mesh,
        scratch_types=[
            pltpu.VMEM((C * 3,), jnp.float32),
            pltpu.VMEM((NR, C), jnp.int32),
            pltpu.VMEM((NR, C), jnp.int32),
            pltpu.VMEM((N_LEVELS * 3, C), jnp.float32),
            pltpu.VMEM((NR, C, 8), jnp.float32),
            pltpu.VMEM((C * NCOL,), jnp.float32),
            pltpu.SemaphoreType.DMA((N_LEVELS,)),
        ],
        compiler_params=pltpu.CompilerParams(
            needs_layout_passes=False, use_tc_tiling_on_sc=False),
    )
    return k(xyzf, tbl).reshape(n, NCOL)


# speedup vs baseline: 146.2227x; 1.0052x over previous
import numpy as np
import jax
import jax.numpy as jnp
from jax import lax
from jax.experimental import pallas as pl
from jax.experimental.pallas import tpu as pltpu
from jax.experimental.pallas import tpu_sc as plsc

N_LEVELS = 16
F = 2
LOG2_T = 19
T = 2 ** LOG2_T
BASE_RES = 16
PER_LEVEL_SCALE = 1.3819129
PRIMES = (1, 2654435761, 805459861)

NW = 32
C = 64
NCOL = 3 + N_LEVELS * F
NR = N_LEVELS * 8


def _levels():
    out = []
    for l in range(N_LEVELS):
        res = int(np.floor(BASE_RES * (PER_LEVEL_SCALE ** l)))
        stride = res + 1
        out.append((res, stride, stride ** 3 <= T, l * T))
    return out


LEVELS = _levels()


def _grid_coords(x, y, z, res):
    rf = jnp.float32(res)
    sx, sy, sz = x * rf, y * rf, z * rf
    ix = sx.astype(jnp.int32)
    iy = sy.astype(jnp.int32)
    iz = sz.astype(jnp.int32)
    return sx, sy, sz, ix, iy, iz


RELAYOUT_CH = 16384


def _relayout_body(tsrc, tdst, src_loc, dst_loc):
    wid = lax.axis_index("s") * 2 + lax.axis_index("c")
    span = tsrc.shape[0] // NW
    base = wid * span
    iota = lax.iota(jnp.int32, 16)
    io2 = iota * 2

    def chunk(c, carry):
        off = base + c * RELAYOUT_CH
        pltpu.sync_copy(tsrc.at[pl.ds(off, RELAYOUT_CH)], src_loc)

        def tile(t, c2):
            tb = t * 256
            for k in range(8):
                f0 = src_loc[pl.ds(tb + k * 16, 16)]
                f1 = src_loc[pl.ds(tb + 128 + k * 16, 16)]
                di = io2 + (tb + k * 32)
                plsc.store_scatter(dst_loc, [di], f0)
                plsc.store_scatter(dst_loc, [di + 1], f1)
            return c2

        lax.fori_loop(0, RELAYOUT_CH // 256, tile, 0)
        pltpu.sync_copy(dst_loc, tdst.at[pl.ds(off, RELAYOUT_CH)])
        return carry

    lax.fori_loop(0, span // RELAYOUT_CH, chunk, 0)


def _body(xyzf, tbl, out, xyz_loc, idx_buf, fv_buf, w_buf, rows, obuf, sems):
    wid = lax.axis_index("s") * 2 + lax.axis_index("c")
    npts = xyzf.shape[0] // 3
    per_w = npts // NW
    nblk = per_w // C
    iota = lax.iota(jnp.int32, 16)

    def block(b, carry):
        base = wid * per_w + b * C

        pltpu.sync_copy(xyzf.at[pl.ds(base * 3, C * 3)], xyz_loc)

        for l, (res, stride, dense, lbase) in enumerate(LEVELS):

            def grp_a(g, c2, res=res, stride=stride, dense=dense,
                      lbase=lbase, l=l):
                for so in (0, 16):
                    o = g * 32 + so
                    p3 = (iota + o) * 3
                    x = plsc.load_gather(xyz_loc, [p3])
                    y = plsc.load_gather(xyz_loc, [p3 + 1])
                    z = plsc.load_gather(xyz_loc, [p3 + 2])
                    sx, sy, sz, ix, iy, iz = _grid_coords(x, y, z, res)
                    w_buf[l * 3, pl.ds(o, 16)] = sx - ix.astype(jnp.float32)
                    w_buf[l * 3 + 1, pl.ds(o, 16)] = sy - iy.astype(jnp.float32)
                    w_buf[l * 3 + 2, pl.ds(o, 16)] = sz - iz.astype(jnp.float32)
                    if dense:
                        s2 = stride * stride
                        b000 = ix + iy * stride + iz * s2 + lbase
                        for corner in range(8):
                            off = ((corner & 1) + ((corner >> 1) & 1) * stride
                                   + ((corner >> 2) & 1) * s2)
                            e = b000 + off
                            r2, oc = divmod(l * 8 + corner, 2)
                            idx_buf[r2, pl.ds(oc * C + o, 16)] = e >> 2
                            fv_buf[r2, pl.ds(oc * C + o, 16)] = (e & 3) << 1
                    else:
                        ux = ix.astype(jnp.uint32)
                        uy = iy.astype(jnp.uint32)
                        uz = iz.astype(jnp.uint32)
                        p1 = jnp.uint32(PRIMES[1])
                        p2 = jnp.uint32(PRIMES[2])
                        hy0 = uy * p1
                        hy1 = hy0 + p1
                        hz0 = uz * p2
                        hz1 = hz0 + p2
                        hx1 = ux + jnp.uint32(1)
                        mask = jnp.uint32(T - 1)
                        for corner in range(8):
                            hx = hx1 if (corner & 1) else ux
                            hy = hy1 if (corner & 2) else hy0
                            hz = hz1 if (corner & 4) else hz0
                            h = (hx ^ hy ^ hz) & mask
                            e = h.astype(jnp.int32) + lbase
                            r2, oc = divmod(l * 8 + corner, 2)
                            idx_buf[r2, pl.ds(oc * C + o, 16)] = e >> 2
                            fv_buf[r2, pl.ds(oc * C + o, 16)] = (e & 3) << 1
                return c2

            lax.fori_loop(0, C // 32, grp_a, 0)
            for j in range(4):
                r2 = l * 4 + j
                pltpu.async_copy(tbl.at[idx_buf.at[r2]], rows.at[r2],
                                 sems.at[l])

        for l, (res, stride, dense, lbase) in enumerate(LEVELS):
            for j in range(4):
                r2 = l * 4 + j
                pltpu.make_async_copy(
                    tbl.at[idx_buf.at[r2]], rows.at[r2], sems.at[l]).wait()

            def grp_c(g, c2, l=l):
                for so in (0, 16):
                    o = g * 32 + so
                    pv = iota + o
                    pcol = pv * NCOL
                    fx = w_buf[l * 3, pl.ds(o, 16)]
                    fy = w_buf[l * 3 + 1, pl.ds(o, 16)]
                    fz = w_buf[l * 3 + 2, pl.ds(o, 16)]
                    gx, gy, gz = 1.0 - fx, 1.0 - fy, 1.0 - fz
                    wxy = (gx * gy, fx * gy, gx * fy, fx * fy)
                    acc0 = acc1 = None
                    for corner in range(8):
                        wc = wxy[corner & 3] * (fz if (corner & 4) else gz)
                        r2, oc = divmod(l * 8 + corner, 2)
                        rv = jnp.full((16,), r2, jnp.int32)
                        pv2 = pv + oc * C
                        fv = fv_buf[r2, pl.ds(oc * C + o, 16)]
                        f0 = plsc.load_gather(rows, [rv, pv2, fv])
                        f1 = plsc.load_gather(rows, [rv, pv2, fv + 1])
                        if corner == 0:
                            acc0, acc1 = f0 * wc, f1 * wc
                        else:
                            acc0, acc1 = acc0 + f0 * wc, acc1 + f1 * wc
                    plsc.store_scatter(obuf, [pcol + (3 + 2 * l)], acc0)
                    plsc.store_scatter(obuf, [pcol + (4 + 2 * l)], acc1)
                return c2

            lax.fori_loop(0, C // 32, grp_c, 0)

        def grp_x(g, c2):
            o = g * 16
            pv = iota + o
            p3 = pv * 3
            pcol = pv * NCOL
            x = plsc.load_gather(xyz_loc, [p3])
            y = plsc.load_gather(xyz_loc, [p3 + 1])
            z = plsc.load_gather(xyz_loc, [p3 + 2])
            plsc.store_scatter(obuf, [pcol], x * 2.0 - 1.0)
            plsc.store_scatter(obuf, [pcol + 1], y * 2.0 - 1.0)
            plsc.store_scatter(obuf, [pcol + 2], z * 2.0 - 1.0)
            return c2

        lax.fori_loop(0, C // 16, grp_x, 0)

        pltpu.sync_copy(obuf, out.at[pl.ds(base * NCOL, C * NCOL)])
        return carry

    lax.fori_loop(0, nblk, block, 0)


def kernel(xyz, table):
    n = xyz.shape[0]
    nw = N_LEVELS * T * F
    tnative = (table.reshape(N_LEVELS, T // 128, 128, F)
               .transpose(0, 1, 3, 2).reshape(nw))
    xyzf = xyz.reshape(n * 3)
    mesh = plsc.VectorSubcoreMesh(core_axis_name="c", subcore_axis_name="s")
    k1 = pl.kernel(
        _relayout_body,
        out_type=jax.ShapeDtypeStruct((nw,), jnp.float32),
        mesh=mesh,
        scratch_types=[
            pltpu.VMEM((RELAYOUT_CH,), jnp.float32),
            pltpu.VMEM((RELAYOUT_CH,), jnp.float32),
        ],
        compiler_params=pltpu.CompilerParams(
            needs_layout_passes=False, use_tc_tiling_on_sc=False),
    )
    tbl = k1(tnative).reshape(nw // 8, 8)
    k = pl.kernel(
        _body,
        out_type=jax.ShapeDtypeStruct((n * NCOL,), jnp.float32),
        mesh=mesh,
        scratch_types=[
            pltpu.VMEM((C * 3,), jnp.float32),
            pltpu.VMEM((NR // 2, C * 2), jnp.int32),
            pltpu.VMEM((NR // 2, C * 2), jnp.int32),
            pltpu.VMEM((N_LEVELS * 3, C), jnp.float32),
            pltpu.VMEM((NR // 2, C * 2, 8), jnp.float32),
            pltpu.VMEM((C * NCOL,), jnp.float32),
            pltpu.SemaphoreType.DMA((N_LEVELS,)),
        ],
        compiler_params=pltpu.CompilerParams(
            needs_layout_passes=False, use_tc_tiling_on_sc=False),
    )
    return k(xyzf, tbl).reshape(n, NCOL)


# speedup vs baseline: 146.6805x; 1.0031x over previous
import numpy as np
import jax
import jax.numpy as jnp
from jax import lax
from jax.experimental import pallas as pl
from jax.experimental.pallas import tpu as pltpu
from jax.experimental.pallas import tpu_sc as plsc

N_LEVELS = 16
F = 2
LOG2_T = 19
T = 2 ** LOG2_T
BASE_RES = 16
PER_LEVEL_SCALE = 1.3819129
PRIMES = (1, 2654435761, 805459861)

NW = 32
C = 64
NCOL = 3 + N_LEVELS * F
NR = N_LEVELS * 8


def _levels():
    out = []
    for l in range(N_LEVELS):
        res = int(np.floor(BASE_RES * (PER_LEVEL_SCALE ** l)))
        stride = res + 1
        out.append((res, stride, stride ** 3 <= T, l * T))
    return out


LEVELS = _levels()


def _grid_coords(x, y, z, res):
    rf = jnp.float32(res)
    sx, sy, sz = x * rf, y * rf, z * rf
    ix = sx.astype(jnp.int32)
    iy = sy.astype(jnp.int32)
    iz = sz.astype(jnp.int32)
    return sx, sy, sz, ix, iy, iz


RELAYOUT_CH = 16384


def _relayout_body(tsrc, tdst, src_loc, dst_loc):
    wid = lax.axis_index("s") * 2 + lax.axis_index("c")
    span = tsrc.shape[0] // NW
    base = wid * span
    iota = lax.iota(jnp.int32, 16)
    io2 = iota * 2

    def chunk(c, carry):
        off = base + c * RELAYOUT_CH
        pltpu.sync_copy(tsrc.at[pl.ds(off, RELAYOUT_CH)], src_loc)

        def tile(t, c2):
            tb = t * 256
            for k in range(8):
                f0 = src_loc[pl.ds(tb + k * 16, 16)]
                f1 = src_loc[pl.ds(tb + 128 + k * 16, 16)]
                di = io2 + (tb + k * 32)
                plsc.store_scatter(dst_loc, [di], f0)
                plsc.store_scatter(dst_loc, [di + 1], f1)
            return c2

        lax.fori_loop(0, RELAYOUT_CH // 256, tile, 0)
        pltpu.sync_copy(dst_loc, tdst.at[pl.ds(off, RELAYOUT_CH)])
        return carry

    lax.fori_loop(0, span // RELAYOUT_CH, chunk, 0)


H0 = tuple(enumerate(LEVELS))[:8]
H1 = tuple(enumerate(LEVELS))[8:]


def _body(xyzf, tbl, out, xyz_loc, idx_buf, fv_buf, w_buf, rows, obuf,
          sems, sem_x):
    wid = lax.axis_index("s") * 2 + lax.axis_index("c")
    npts = xyzf.shape[0] // 3
    per_w = npts // NW
    nblk = per_w // C
    iota = lax.iota(jnp.int32, 16)
    base0 = wid * per_w

    def phase_a(levels, pb):
        pbv = jnp.zeros((16,), jnp.int32) + pb
        for l, (res, stride, dense, lbase) in levels:

            def grp_a(g, c2, res=res, stride=stride, dense=dense,
                      lbase=lbase, l=l):
                for so in (0, 16):
                    o = g * 32 + so
                    p3 = (iota + o) * 3
                    x = plsc.load_gather(xyz_loc, [pbv, p3])
                    y = plsc.load_gather(xyz_loc, [pbv, p3 + 1])
                    z = plsc.load_gather(xyz_loc, [pbv, p3 + 2])
                    sx, sy, sz, ix, iy, iz = _grid_coords(x, y, z, res)
                    w_buf[l * 3, pl.ds(o, 16)] = sx - ix.astype(jnp.float32)
                    w_buf[l * 3 + 1, pl.ds(o, 16)] = sy - iy.astype(jnp.float32)
                    w_buf[l * 3 + 2, pl.ds(o, 16)] = sz - iz.astype(jnp.float32)
                    if dense:
                        s2 = stride * stride
                        b000 = ix + iy * stride + iz * s2 + lbase
                        for corner in range(8):
                            off = ((corner & 1) + ((corner >> 1) & 1) * stride
                                   + ((corner >> 2) & 1) * s2)
                            e = b000 + off
                            r4, oc = divmod(l * 8 + corner, 4)
                            idx_buf[r4, pl.ds(oc * C + o, 16)] = e >> 2
                            fv_buf[r4, pl.ds(oc * C + o, 16)] = (e & 3) << 1
                    else:
                        ux = ix.astype(jnp.uint32)
                        uy = iy.astype(jnp.uint32)
                        uz = iz.astype(jnp.uint32)
                        p1 = jnp.uint32(PRIMES[1])
                        p2 = jnp.uint32(PRIMES[2])
                        hy0 = uy * p1
                        hy1 = hy0 + p1
                        hz0 = uz * p2
                        hz1 = hz0 + p2
                        hx1 = ux + jnp.uint32(1)
                        mask = jnp.uint32(T - 1)
                        for corner in range(8):
                            hx = hx1 if (corner & 1) else ux
                            hy = hy1 if (corner & 2) else hy0
                            hz = hz1 if (corner & 4) else hz0
                            h = (hx ^ hy ^ hz) & mask
                            e = h.astype(jnp.int32) + lbase
                            r4, oc = divmod(l * 8 + corner, 4)
                            idx_buf[r4, pl.ds(oc * C + o, 16)] = e >> 2
                            fv_buf[r4, pl.ds(oc * C + o, 16)] = (e & 3) << 1
                return c2

            lax.fori_loop(0, C // 32, grp_a, 0)
            for j in range(2):
                r4 = l * 2 + j
                pltpu.async_copy(tbl.at[idx_buf.at[r4]], rows.at[r4],
                                 sems.at[l])

    def drain_half(levels):
        for l, _ in levels:
            for j in range(2):
                r4 = l * 2 + j
                pltpu.make_async_copy(
                    tbl.at[idx_buf.at[r4]], rows.at[r4], sems.at[l]).wait()

    def phase_c(levels):
        for l, _ in levels:

            def grp_c(g, c2, l=l):
                for so in (0, 16):
                    o = g * 32 + so
                    pv = iota + o
                    pcol = pv * NCOL
                    fx = w_buf[l * 3, pl.ds(o, 16)]
                    fy = w_buf[l * 3 + 1, pl.ds(o, 16)]
                    fz = w_buf[l * 3 + 2, pl.ds(o, 16)]
                    gx, gy, gz = 1.0 - fx, 1.0 - fy, 1.0 - fz
                    wxy = (gx * gy, fx * gy, gx * fy, fx * fy)
                    acc0 = acc1 = None
                    for corner in range(8):
                        wc = wxy[corner & 3] * (fz if (corner & 4) else gz)
                        r4, oc = divmod(l * 8 + corner, 4)
                        rv = jnp.full((16,), r4, jnp.int32)
                        pv4 = pv + oc * C
                        fv = fv_buf[r4, pl.ds(oc * C + o, 16)]
                        f0 = plsc.load_gather(rows, [rv, pv4, fv])
                        f1 = plsc.load_gather(rows, [rv, pv4, fv + 1])
                        if corner == 0:
                            acc0, acc1 = f0 * wc, f1 * wc
                        else:
                            acc0, acc1 = acc0 + f0 * wc, acc1 + f1 * wc
                    plsc.store_scatter(obuf, [pcol + (3 + 2 * l)], acc0)
                    plsc.store_scatter(obuf, [pcol + (4 + 2 * l)], acc1)
                return c2

            lax.fori_loop(0, C // 32, grp_c, 0)

    def finish_xyz(pb):
        pbv = jnp.zeros((16,), jnp.int32) + pb

        def grp_x(g, c2):
            o = g * 16
            pv = iota + o
            p3 = pv * 3
            pcol = pv * NCOL
            x = plsc.load_gather(xyz_loc, [pbv, p3])
            y = plsc.load_gather(xyz_loc, [pbv, p3 + 1])
            z = plsc.load_gather(xyz_loc, [pbv, p3 + 2])
            plsc.store_scatter(obuf, [pcol], x * 2.0 - 1.0)
            plsc.store_scatter(obuf, [pcol + 1], y * 2.0 - 1.0)
            plsc.store_scatter(obuf, [pcol + 2], z * 2.0 - 1.0)
            return c2

        lax.fori_loop(0, C // 16, grp_x, 0)

    pltpu.sync_copy(xyzf.at[pl.ds(base0 * 3, C * 3)], xyz_loc.at[0])
    phase_a(H0, 0)

    def block(b, carry):
        pb = b & 1
        base = base0 + b * C
        bn = jnp.minimum(b + 1, nblk - 1)
        basen = base0 + bn * C

        phase_a(H1, pb)
        xcp = pltpu.async_copy(
            xyzf.at[pl.ds(basen * 3, C * 3)], xyz_loc.at[1 - pb], sem_x)
        drain_half(H0)
        phase_c(H0)
        finish_xyz(pb)
        pltpu.make_async_copy(
            xyzf.at[pl.ds(basen * 3, C * 3)], xyz_loc.at[1 - pb], sem_x).wait()
        phase_a(H0, 1 - pb)
        drain_half(H1)
        phase_c(H1)
        pltpu.sync_copy(obuf, out.at[pl.ds(base * NCOL, C * NCOL)])
        return carry

    lax.fori_loop(0, nblk, block, 0)

    drain_half(H0)


def kernel(xyz, table):
    n = xyz.shape[0]
    nw = N_LEVELS * T * F
    tnative = (table.reshape(N_LEVELS, T // 128, 128, F)
               .transpose(0, 1, 3, 2).reshape(nw))
    xyzf = xyz.reshape(n * 3)
    mesh = plsc.VectorSubcoreMesh(core_axis_name="c", subcore_axis_name="s")
    k1 = pl.kernel(
        _relayout_body,
        out_type=jax.ShapeDtypeStruct((nw,), jnp.float32),
        mesh=mesh,
        scratch_types=[
            pltpu.VMEM((RELAYOUT_CH,), jnp.float32),
            pltpu.VMEM((RELAYOUT_CH,), jnp.float32),
        ],
        compiler_params=pltpu.CompilerParams(
            needs_layout_passes=False, use_tc_tiling_on_sc=False),
    )
    tbl = k1(tnative).reshape(nw // 8, 8)
    k = pl.kernel(
        _body,
        out_type=jax.ShapeDtypeStruct((n * NCOL,), jnp.float32),
        mesh=mesh,
        scratch_types=[
            pltpu.VMEM((2, C * 3), jnp.float32),
            pltpu.VMEM((NR // 4, C * 4), jnp.int32),
            pltpu.VMEM((NR // 4, C * 4), jnp.int32),
            pltpu.VMEM((N_LEVELS * 3, C), jnp.float32),
            pltpu.VMEM((NR // 4, C * 4, 8), jnp.float32),
            pltpu.VMEM((C * NCOL,), jnp.float32),
            pltpu.SemaphoreType.DMA((N_LEVELS,)),
            pltpu.SemaphoreType.DMA,
        ],
        compiler_params=pltpu.CompilerParams(
            needs_layout_passes=False, use_tc_tiling_on_sc=False),
    )
    return k(xyzf, tbl).reshape(n, NCOL)
